# baseline (speedup 1.0000x reference)
import functools

import jax
import jax.numpy as jnp
from jax import lax
from jax.experimental import pallas as pl
from jax.experimental.pallas import tpu as pltpu
from jax.experimental.pallas import tpu_sc as plsc

NBUF = 2
BCAP = 24
SCAP = 96
NBK = 128


def _make_sc_gather(B, C, V):
    info = plsc.get_sparse_core_info()
    NC, NS, L = info.num_cores, info.num_subcores, info.num_lanes
    NW = NC * NS
    NT = (V + 127) // 128
    NP = (NT + 1) // 2
    TPW = (NP + NW - 1) // NW
    OUTER = (TPW + NBUF - 1) // NBUF
    last_start = 128 * (NT - 2)
    mesh = plsc.VectorSubcoreMesh(core_axis_name="c", subcore_axis_name="s")
    thresh = SCAP - BCAP - 2 * L

    @functools.partial(
        pl.kernel,
        mesh=mesh,
        out_type=jax.ShapeDtypeStruct((B + 8, 2 * C), jnp.float32),
        scratch_types=[
            pltpu.VMEM((B,), jnp.int32),
            pltpu.VMEM((NBK, BCAP), jnp.int32),
            pltpu.VMEM((NBK, BCAP), jnp.int32),
            pltpu.VMEM((NBK,), jnp.int32),
            *[pltpu.VMEM((C, 4 * C), jnp.float32) for _ in range(NBUF)],
            pltpu.VMEM((SCAP, 2 * C), jnp.float32),
            pltpu.VMEM((SCAP,), jnp.int32),
            pltpu.VMEM((2 * L,), jnp.int32),
            pltpu.VMEM((2 * L,), jnp.int32),
            *[pltpu.SemaphoreType.DMA for _ in range(NBUF)],
            pltpu.SemaphoreType.DMA,
        ],
        compiler_params=pltpu.CompilerParams(needs_layout_passes=False),
    )
    def gather(ids_hbm, t_hbm, out_hbm, ids_v, bids, bpos, bcnt, *rest):
        chunks = rest[:NBUF]
        rows_v, spos_v, tmp_i, tmp_p = rest[NBUF : NBUF + 4]
        dsems = rest[NBUF + 4 : 2 * NBUF + 4]
        ssem = rest[2 * NBUF + 4]
        w = lax.axis_index("s") * NC + lax.axis_index("c")
        lane = lax.iota(jnp.int32, L)
        lane0 = lane == 0
        zeros = jnp.zeros((L,), jnp.int32)

        pltpu.sync_copy(ids_hbm, ids_v)

        def init_cnt(i, c):
            bcnt[pl.ds(i * L, L)] = zeros
            return c

        lax.fori_loop(0, NBK // L, init_cnt, 0)

        def init_spos(i, c):
            spos_v[pl.ds(i * L, L)] = zeros + B
            return c

        lax.fori_loop(0, SCAP // L, init_spos, 0)

        def scan_vreg(i, c):
            v = ids_v[pl.ds(i * L, L)]
            t = lax.shift_right_logical(v, 8)
            mine = (t & (NW - 1)) == w
            n = plsc.all_reduce_population_count(mine)[0]

            @pl.when(n > 0)
            def _():
                plsc.store_compressed(tmp_i.at[pl.ds(0, L)], v, mask=mine)
                plsc.store_compressed(tmp_p.at[pl.ds(0, L)], lane + i * L, mask=mine)

                def put(k, c2):
                    sid = tmp_i[pl.ds(k, L)][0]
                    sp = tmp_p[pl.ds(k, L)][0]
                    bk = lax.shift_right_logical(sid, 13)
                    cb = plsc.load_gather(bcnt, [zeros + bk])[0]

                    @pl.when(cb < BCAP)
                    def _():
                        plsc.store_scatter(
                            bids, [zeros + bk, zeros + cb], zeros + sid,
                            mask=lane0,
                        )
                        plsc.store_scatter(
                            bpos, [zeros + bk, zeros + cb], zeros + sp,
                            mask=lane0,
                        )

                    plsc.store_scatter(
                        bcnt, [zeros + bk], zeros + cb + 1, mask=lane0
                    )
                    return c2

                lax.fori_loop(0, n, put, 0)

            return c

        lax.fori_loop(0, B // L, scan_vreg, 0)

        def tile_start(pnum):
            return pl.multiple_of(jnp.minimum(pnum * 256, last_start), 128)

        def issue(ci, b):
            pnum = ci * NW + w
            start = tile_start(pnum)
            pltpu.async_copy(
                t_hbm.at[:, pl.ds(start, 256)], chunks[b], dsems[b]
            )

        for b in range(NBUF):
            issue(jnp.int32(b), b)

        def extract_item(sid, sp, start, s, b):
            wl = sid - start
            for q in range(C // L):
                x = plsc.load_gather(chunks[b], [lane + q * L, zeros + wl])
                rows_v[s, pl.ds(q * L, L)] = x
            plsc.store_scatter(spos_v, [zeros + s], zeros + sp, mask=lane0)
            return s + 1

        def flush(s, always=False):
            do = (s > 0) if always else (s >= thresh)

            @pl.when(do)
            def _():
                pltpu.async_copy(rows_v, out_hbm.at[spos_v], ssem).wait()

                def reset(i, c):
                    spos_v[pl.ds(i * L, L)] = zeros + B
                    return c

                lax.fori_loop(0, SCAP // L, reset, 0)

            return jnp.where(do, 0, s)

        def slow_path(ci, tnum, start, b, s, active):
            nv = jnp.where(active, B // L, 0)

            def svreg(i, s2):
                v = ids_v[pl.ds(i * L, L)]
                m = lax.shift_right_logical(v, 8) == tnum
                n = plsc.all_reduce_population_count(m)[0]

                @pl.when(n > 0)
                def _():
                    plsc.store_compressed(tmp_i.at[pl.ds(0, L)], v, mask=m)
                    plsc.store_compressed(
                        tmp_p.at[pl.ds(0, L)], lane + i * L, mask=m
                    )

                def put(k, s3):
                    sid = tmp_i[pl.ds(k, L)][0]
                    sp = tmp_p[pl.ds(k, L)][0]
                    return extract_item(sid, sp, start, s3, b)

                s2 = lax.fori_loop(0, n, put, s2)
                return flush(s2)

            return lax.fori_loop(0, nv, svreg, s)

        def outer_body(co, s):
            for b in range(NBUF):
                ci = co * NBUF + b
                live = ci < TPW

                @pl.when(live)
                def _():
                    pltpu.make_async_copy(
                        t_hbm.at[:, pl.ds(0, 256)], chunks[b], dsems[b]
                    ).wait()

                tnum = ci * NW + w
                start = tile_start(tnum)
                nc = plsc.load_gather(bcnt, [zeros + ci])[0]
                nfast = jnp.minimum(
                    jnp.where(live, nc, 0), jnp.int32(BCAP)
                )

                def fast(k, s2):
                    sid = plsc.load_gather(bids, [zeros + ci, zeros + k])[0]
                    sp = plsc.load_gather(bpos, [zeros + ci, zeros + k])[0]
                    return extract_item(sid, sp, start, s2, b)

                s = lax.fori_loop(0, nfast, fast, s)
                s = flush(s)
                s = slow_path(ci, tnum, start, b, s, live & (nc > BCAP))

                @pl.when(ci + NBUF < TPW)
                def _():
                    issue(ci + NBUF, b)

            return s

        s = lax.fori_loop(0, OUTER, outer_body, jnp.int32(0))
        flush(s, always=True)

    return gather


def kernel(item_ids, emb_weight):
    B, = item_ids.shape
    V, D = emb_weight.shape
    ids = item_ids.astype(jnp.int32)
    wide = _make_sc_gather(B, D, V)(ids, emb_weight.T)
    return wide[:B, :D]


# speedup vs baseline: 1.4777x; 1.4777x over previous
import functools

import jax
import jax.numpy as jnp
from jax import lax
from jax.experimental import pallas as pl
from jax.experimental.pallas import tpu as pltpu
from jax.experimental.pallas import tpu_sc as plsc

WIN = 2048
BCAP = 64
SCAP = 128


def _make_sc_gather(B, C, V):
    info = plsc.get_sparse_core_info()
    NC, NS, L = info.num_cores, info.num_subcores, info.num_lanes
    NW = NC * NS
    NT = (V + 127) // 128
    NWIN = -(-V // WIN)
    WPW = -(-NWIN // NW)
    mesh = plsc.VectorSubcoreMesh(core_axis_name="c", subcore_axis_name="s")
    strip_clamp = 128 * (NT - WIN // 128)
    thresh = SCAP - BCAP

    @functools.partial(
        pl.kernel,
        mesh=mesh,
        out_type=jax.ShapeDtypeStruct((B + 8, 2 * C), jnp.float32),
        scratch_types=[
            pltpu.VMEM((B,), jnp.int32),
            pltpu.VMEM((WPW, BCAP), jnp.int32),
            pltpu.VMEM((WPW, BCAP), jnp.int32),
            pltpu.VMEM((WPW,), jnp.int32),
            pltpu.VMEM((8, WIN), jnp.float32),
            pltpu.VMEM((8, WIN), jnp.float32),
            pltpu.VMEM((SCAP, 2 * C), jnp.float32),
            pltpu.VMEM((SCAP,), jnp.int32),
            pltpu.VMEM((2 * L,), jnp.int32),
            pltpu.VMEM((2 * L,), jnp.int32),
            pltpu.SemaphoreType.DMA,
            pltpu.SemaphoreType.DMA,
            pltpu.SemaphoreType.DMA,
        ],
        compiler_params=pltpu.CompilerParams(needs_layout_passes=False),
    )
    def gather(
        ids_hbm, t_hbm, out_hbm, ids_v, bids, bpos, bcnt,
        buf0, buf1, rows_v, spos_v, tmp_i, tmp_p, sem0, sem1, ssem,
    ):
        chunks = (buf0, buf1)
        dsems = (sem0, sem1)
        w = lax.axis_index("s") * NC + lax.axis_index("c")
        lane = lax.iota(jnp.int32, L)
        lane0 = lane == 0
        lo8 = lane < 8
        zeros = jnp.zeros((L,), jnp.int32)

        pltpu.sync_copy(ids_hbm, ids_v)

        def init_cnt(i, c):
            bcnt[pl.ds(i * L, L)] = zeros
            return c

        lax.fori_loop(0, WPW // L, init_cnt, 0)

        def init_spos(i, c):
            spos_v[pl.ds(i * L, L)] = zeros + B
            return c

        lax.fori_loop(0, SCAP // L, init_spos, 0)

        def scan_vreg(i, c):
            v = ids_v[pl.ds(i * L, L)]
            t = lax.shift_right_logical(v, 11)
            mine = (t & (NW - 1)) == w
            n = plsc.all_reduce_population_count(mine)[0]

            @pl.when(n > 0)
            def _():
                plsc.store_compressed(tmp_i.at[pl.ds(0, L)], v, mask=mine)
                plsc.store_compressed(
                    tmp_p.at[pl.ds(0, L)], lane + i * L, mask=mine
                )

                def put(k, c2):
                    sid = tmp_i[pl.ds(k, L)][0]
                    sp = tmp_p[pl.ds(k, L)][0]
                    bk = lax.shift_right_logical(sid, 16)
                    cb = plsc.load_gather(bcnt, [zeros + bk])[0]

                    @pl.when(cb < BCAP)
                    def _():
                        plsc.store_scatter(
                            bids, [zeros + bk, zeros + cb], zeros + sid,
                            mask=lane0,
                        )
                        plsc.store_scatter(
                            bpos, [zeros + bk, zeros + cb], zeros + sp,
                            mask=lane0,
                        )

                    plsc.store_scatter(
                        bcnt, [zeros + bk], zeros + cb + 1, mask=lane0
                    )
                    return c2

                lax.fori_loop(0, n, put, 0)

            return c

        lax.fori_loop(0, B // L, scan_vreg, 0)

        def win_start(wo):
            wnum = wo * NW + w
            return pl.multiple_of(
                jnp.minimum(wnum * WIN, strip_clamp), 128
            )

        def issue(wo, a, p):
            pltpu.async_copy(
                t_hbm.at[pl.ds(8 * a, 8), pl.ds(win_start(wo), WIN)],
                chunks[p],
                dsems[p],
            )

        def wait_buf(p):
            pltpu.make_async_copy(
                t_hbm.at[pl.ds(0, 8), pl.ds(0, WIN)], chunks[p], dsems[p]
            ).wait()

        issue(jnp.int32(0), 0, 0)
        issue(jnp.int32(0), 1, 1)

        def flush(s, limit):
            do = s >= limit

            @pl.when(do)
            def _():
                pltpu.async_copy(rows_v, out_hbm.at[spos_v], ssem).wait()

                def reset(i, c):
                    spos_v[pl.ds(i * L, L)] = zeros + B
                    return c

                lax.fori_loop(0, SCAP // L, reset, 0)

            return jnp.where(do, 0, s)

        def window_body(wo, s):
            s = flush(s, thresh)
            start = win_start(wo)
            wnum = wo * NW + w
            nc = plsc.load_gather(bcnt, [zeros + wo])[0]
            nf = jnp.minimum(nc, jnp.int32(BCAP))

            for a in range(8):
                p = a & 1
                wait_buf(p)

                def strip_item(k, c2, a=a, p=p):
                    sid = plsc.load_gather(bids, [zeros + wo, zeros + k])[0]
                    wl = sid - start
                    x = plsc.load_gather(
                        chunks[p], [lane & 7, zeros + wl], mask=lo8
                    )
                    plsc.store_scatter(
                        rows_v,
                        [zeros + s + k, (lane & 7) + 8 * a],
                        x,
                        mask=lo8,
                    )
                    if a == 0:
                        sp = plsc.load_gather(
                            bpos, [zeros + wo, zeros + k]
                        )[0]
                        plsc.store_scatter(
                            spos_v, [zeros + s + k], zeros + sp, mask=lane0
                        )
                    return c2

                lax.fori_loop(0, nf, strip_item, 0)

                nxt = wo * 8 + a + 2
                issue(lax.shift_right_logical(nxt, 3), (a + 2) & 7, p)

            s = s + nf

            over = nc > BCAP

            @pl.when(over)
            def _():
                wait_buf(0)

            def srescan(i, carry):
                cnt, s2 = carry
                v = ids_v[pl.ds(i * L, L)]
                m = lax.shift_right_logical(v, 11) == wnum
                n = plsc.all_reduce_population_count(m)[0]

                @pl.when(n > 0)
                def _():
                    plsc.store_compressed(tmp_i.at[pl.ds(0, L)], v, mask=m)
                    plsc.store_compressed(
                        tmp_p.at[pl.ds(0, L)], lane + i * L, mask=m
                    )

                def sput(k, c3):
                    cnt3, s3 = c3
                    s3 = flush(s3, SCAP - 1)
                    sid = tmp_i[pl.ds(k, L)][0]
                    sp = tmp_p[pl.ds(k, L)][0]
                    go = cnt3 >= BCAP

                    @pl.when(go)
                    def _():
                        base = pl.multiple_of(
                            lax.shift_right_logical(sid, 7) * 128, 128
                        )
                        for aa in range(8):
                            pltpu.async_copy(
                                t_hbm.at[
                                    pl.ds(8 * aa, 8), pl.ds(base, 128)
                                ],
                                chunks[0].at[:, pl.ds(128 * aa, 128)],
                                dsems[0],
                            )
                        pltpu.make_async_copy(
                            t_hbm.at[pl.ds(0, 8), pl.ds(0, 1024)],
                            chunks[0].at[:, pl.ds(0, 1024)],
                            dsems[0],
                        ).wait()
                        wlp = sid - base
                        for q in range(2 * C // L):
                            cvec = lane + q * L
                            x = plsc.load_gather(
                                chunks[0],
                                [
                                    cvec & 7,
                                    lax.shift_right_logical(cvec, 3) * 128
                                    + wlp,
                                ],
                            )
                            rows_v[s3, pl.ds(q * L, L)] = x
                        plsc.store_scatter(
                            spos_v, [zeros + s3], zeros + sp, mask=lane0
                        )

                    return (cnt3 + 1, jnp.where(go, s3 + 1, s3))

                return lax.fori_loop(0, n, sput, (cnt, s2))

            nv = jnp.where(over, B // L, 0)
            cnt_s = lax.fori_loop(0, nv, srescan, (jnp.int32(0), s))
            s = cnt_s[1]

            @pl.when(over)
            def _():
                issue(wo + 1, 0, 0)

            return s

        s = lax.fori_loop(0, WPW, window_body, jnp.int32(0))
        flush(s, 1)
        wait_buf(0)
        wait_buf(1)

    return gather


def kernel(item_ids, emb_weight):
    B, = item_ids.shape
    V, D = emb_weight.shape
    ids = item_ids.astype(jnp.int32)
    wide = _make_sc_gather(B, D, V)(ids, emb_weight.T)
    return wide[:B, :D]
